# Initial kernel scaffold; baseline (speedup 1.0000x reference)
import jax
import jax.numpy as jnp
from jax import lax
from jax.experimental import pallas as pl
from jax.experimental.pallas import tpu as pltpu
from jax.experimental.pallas import tpu_sc as plsc

N = 10000
E = 320000
D_IN = 128
H = 256
ENC = 256
PC = 35
KS = 15
NET = 3
HALF = 128

BLK = 256
NB = (N + BLK - 1) // BLK

SC_NS = 16
CHUNK = 400
STRIPE = E // SC_NS
NCHUNK = STRIPE // CHUNK

_F32 = jnp.float32


def _make_agg(do_cnt):
  mesh = plsc.VectorSubcoreMesh(core_axis_name="c", subcore_axis_name="s")
  out_type = [jax.ShapeDtypeStruct((NET, 2, N, HALF), _F32)]
  if do_cnt:
    out_type.append(jax.ShapeDtypeStruct((NET, SC_NS, N), _F32))
  scratch = [
      pltpu.VMEM((CHUNK,), jnp.int32),
      pltpu.VMEM((CHUNK,), jnp.int32),
      pltpu.VMEM((CHUNK, HALF), _F32),
      pltpu.VMEM_SHARED((N, HALF), _F32),
      pltpu.SemaphoreType.DMA,
  ]
  if do_cnt:
    scratch.append(pltpu.VMEM((NET, N), _F32))

  def body(z_ref, src_ref, dst_ref, zero_ref, agg_ref, *rest):
    if do_cnt:
      cnt_ref, idx_s, idx_d, rows, acc, sem, cnt_acc = rest
    else:
      idx_s, idx_d, rows, acc, sem = rest
    c = lax.axis_index("c")
    s = lax.axis_index("s")
    lanes = lax.iota(jnp.int32, 16)
    ones16 = jnp.ones((16,), _F32)

    if do_cnt:
      @pl.when(c == 0)
      def _zero_cnt():
        for e in range(NET):
          def zb(i, carry, e=e):
            cnt_acc[e, pl.ds(i * 16, 16)] = jnp.zeros((16,), _F32)
            return carry
          lax.fori_loop(0, N // 16, zb, 0)

    for e in range(NET):
      @pl.when(s == 0)
      def _zero_acc():
        pltpu.sync_copy(zero_ref, acc)
      plsc.subcore_barrier()

      def chunk_body(i, carry, e=e):
        base = s * STRIPE + i * CHUNK
        pltpu.sync_copy(src_ref.at[e].at[pl.ds(base, CHUNK)], idx_s)
        pltpu.sync_copy(dst_ref.at[e].at[pl.ds(base, CHUNK)], idx_d)
        pltpu.async_copy(z_ref.at[e].at[c].at[idx_s], rows, sem).wait()
        pltpu.sync_copy(rows, acc.at[idx_d], add=True)
        if do_cnt:
          @pl.when(c == 0)
          def _count():
            for g in range(CHUNK // 16):
              d16 = idx_d[pl.ds(g * 16, 16)]
              for l in range(16):
                plsc.addupdate_scatter(
                    cnt_acc.at[e], [d16], ones16, mask=lanes == l)
        return carry

      lax.fori_loop(0, NCHUNK, chunk_body, 0)
      plsc.subcore_barrier()

      @pl.when(s == 0)
      def _writeout():
        pltpu.sync_copy(acc, agg_ref.at[e].at[c])
      plsc.subcore_barrier()

    if do_cnt:
      @pl.when(c == 0)
      def _write_cnt():
        for e in range(NET):
          pltpu.sync_copy(cnt_acc.at[e], cnt_ref.at[e].at[s])

  return pl.kernel(body, out_type=tuple(out_type), mesh=mesh,
                   scratch_types=tuple(scratch))


def _k1_body(x_ref, wl_ref, z_ref):
  z_ref[0, 0] = jnp.dot(x_ref[...], wl_ref[0],
                        preferred_element_type=_F32)


def _k1(x, wl_stack, d_in):
  return pl.pallas_call(
      _k1_body,
      grid=(NET, 2, NB),
      in_specs=[
          pl.BlockSpec((BLK, d_in), lambda e, h, i: (i, 0)),
          pl.BlockSpec((1, d_in, HALF), lambda e, h, i: (e, 0, h)),
      ],
      out_specs=pl.BlockSpec((1, 1, BLK, HALF), lambda e, h, i: (e, h, i, 0)),
      out_shape=jax.ShapeDtypeStruct((NET, 2, N, HALF), _F32),
  )(x, wl_stack)


def _combine(agg_ref, cnt_ref, x_ref, wr_ref, bl_ref):
  cnt = jnp.sum(cnt_ref[...], axis=1)
  recip = 1.0 / jnp.maximum(cnt, 1.0)
  x = x_ref[...]
  wr_s = wr_ref[0] + wr_ref[1] + wr_ref[2]
  bl_s = bl_ref[0] + bl_ref[1] + bl_ref[2]
  acc = jnp.dot(x, wr_s, preferred_element_type=_F32) + bl_s[None, :]
  for e in range(NET):
    agg_e = jnp.concatenate([agg_ref[e, 0], agg_ref[e, 1]], axis=1)
    acc = acc + agg_e * recip[e][:, None]
  return jnp.maximum(acc, 0.0)


def _k2_body(agg_ref, cnt_ref, x_ref, wr_ref, bl_ref, wl1_ref,
             x1_ref, z1_ref):
  x1 = _combine(agg_ref, cnt_ref, x_ref, wr_ref, bl_ref)
  x1_ref[...] = x1
  for e in range(NET):
    z = jnp.dot(x1, wl1_ref[e], preferred_element_type=_F32)
    z1_ref[e, 0] = z[:, :HALF]
    z1_ref[e, 1] = z[:, HALF:]


def _k2(agg0, cnt, x, wr_stack, bl_stack, wl1_stack):
  return pl.pallas_call(
      _k2_body,
      grid=(NB,),
      in_specs=[
          pl.BlockSpec((NET, 2, BLK, HALF), lambda i: (0, 0, i, 0)),
          pl.BlockSpec((NET, SC_NS, BLK), lambda i: (0, 0, i)),
          pl.BlockSpec((BLK, D_IN), lambda i: (i, 0)),
          pl.BlockSpec((NET, D_IN, H), lambda i: (0, 0, 0)),
          pl.BlockSpec((NET, H), lambda i: (0, 0)),
          pl.BlockSpec((NET, H, H), lambda i: (0, 0, 0)),
      ],
      out_specs=[
          pl.BlockSpec((BLK, H), lambda i: (i, 0)),
          pl.BlockSpec((NET, 2, BLK, HALF), lambda i: (0, 0, i, 0)),
      ],
      out_shape=[
          jax.ShapeDtypeStruct((N, H), _F32),
          jax.ShapeDtypeStruct((NET, 2, N, HALF), _F32),
      ],
  )(agg0, cnt, x, wr_stack, bl_stack, wl1_stack)


def _k3_body(agg_ref, cnt_ref, x1_ref, wr_ref, bl_ref, wlin_ref, blin_ref,
             y_ref, ps_ref, psq_ref):
  x2 = _combine(agg_ref, cnt_ref, x1_ref, wr_ref, bl_ref)
  y = jnp.dot(x2, wlin_ref[...], preferred_element_type=_F32) + blin_ref[0]
  y_ref[...] = y
  nb = pl.program_id(0)
  row = nb * BLK + lax.broadcasted_iota(jnp.int32, (BLK, 1), 0)
  ym = jnp.where(row < N, y, 0.0)
  ps_ref[0, 0] = jnp.sum(ym, axis=0)
  psq_ref[0, 0] = jnp.sum(ym * ym, axis=0)


def _k3(agg1, cnt, x1, wr1_stack, bl1_stack, w_lin, b_lin2):
  return pl.pallas_call(
      _k3_body,
      grid=(NB,),
      in_specs=[
          pl.BlockSpec((NET, 2, BLK, HALF), lambda i: (0, 0, i, 0)),
          pl.BlockSpec((NET, SC_NS, BLK), lambda i: (0, 0, i)),
          pl.BlockSpec((BLK, H), lambda i: (i, 0)),
          pl.BlockSpec((NET, H, H), lambda i: (0, 0, 0)),
          pl.BlockSpec((NET, H), lambda i: (0, 0)),
          pl.BlockSpec((H, ENC), lambda i: (0, 0)),
          pl.BlockSpec((1, ENC), lambda i: (0, 0)),
      ],
      out_specs=[
          pl.BlockSpec((BLK, ENC), lambda i: (i, 0)),
          pl.BlockSpec((1, 1, ENC), lambda i: (i, 0, 0)),
          pl.BlockSpec((1, 1, ENC), lambda i: (i, 0, 0)),
      ],
      out_shape=[
          jax.ShapeDtypeStruct((N, ENC), _F32),
          jax.ShapeDtypeStruct((NB, 1, ENC), _F32),
          jax.ShapeDtypeStruct((NB, 1, ENC), _F32),
      ],
  )(agg1, cnt, x1, wr1_stack, bl1_stack, w_lin, b_lin2)


def _layer_norm_rows(h, g, b):
  hm = jnp.mean(h, axis=1, keepdims=True)
  hv = jnp.mean(h * h, axis=1, keepdims=True) - hm * hm
  return (h - hm) * lax.rsqrt(hv + 1e-5) * g[None, :] + b[None, :]


def _k4_body(y_ref, ps_ref, psq_ref, bng_ref, bnb_ref,
             pw1_ref, pb1_ref, plg_ref, plb_ref, pw2_ref, pb2_ref,
             kw1x_ref, kw1p_ref, kb1_ref, klg_ref, klb_ref,
             kw2_ref, kb2_ref, pc_ref, ks_ref):
  tot = jnp.sum(ps_ref[...], axis=(0, 1))
  tot2 = jnp.sum(psq_ref[...], axis=(0, 1))
  mu = tot / N
  var = tot2 / N - mu * mu
  inv = lax.rsqrt(var + 1e-5)
  y = y_ref[...]
  xb = (y - mu[None, :]) * (inv * bng_ref[0])[None, :] + bnb_ref[0][None, :]
  h = jnp.maximum(jnp.dot(xb, pw1_ref[...], preferred_element_type=_F32)
                  + pb1_ref[0][None, :], 0.0)
  h = _layer_norm_rows(h, plg_ref[0], plb_ref[0])
  pc = jnp.dot(h, pw2_ref[...], preferred_element_type=_F32) + pb2_ref[0][None, :]
  pc_ref[...] = pc
  h2 = jnp.maximum(jnp.dot(xb, kw1x_ref[...], preferred_element_type=_F32)
                   + jnp.dot(pc, kw1p_ref[...], preferred_element_type=_F32)
                   + kb1_ref[0][None, :], 0.0)
  h2 = _layer_norm_rows(h2, klg_ref[0], klb_ref[0])
  ks_ref[...] = jnp.dot(h2, kw2_ref[...], preferred_element_type=_F32) \
      + kb2_ref[0][None, :]


def _k4(y, ps, psq, args):
  full2 = lambda a: pl.BlockSpec(a.shape, lambda i: tuple(0 for _ in a.shape))
  return pl.pallas_call(
      _k4_body,
      grid=(NB,),
      in_specs=[
          pl.BlockSpec((BLK, ENC), lambda i: (i, 0)),
          pl.BlockSpec((NB, 1, ENC), lambda i: (0, 0, 0)),
          pl.BlockSpec((NB, 1, ENC), lambda i: (0, 0, 0)),
      ] + [full2(a) for a in args],
      out_specs=[
          pl.BlockSpec((BLK, 128), lambda i: (i, 0)),
          pl.BlockSpec((BLK, 128), lambda i: (i, 0)),
      ],
      out_shape=[
          jax.ShapeDtypeStruct((N, 128), _F32),
          jax.ShapeDtypeStruct((N, 128), _F32),
      ],
  )(y, ps, psq, *args)


def kernel(x_note, edge_index_0, edge_index_1, edge_index_2,
           wl0_0, bl0_0, wr0_0, wl0_1, bl0_1, wr0_1, wl0_2, bl0_2, wr0_2,
           wl1_0, bl1_0, wr1_0, wl1_1, bl1_1, wr1_1, wl1_2, bl1_2, wr1_2,
           w_lin, b_lin, bn_gamma, bn_beta,
           pc_w1, pc_b1, pc_ln_g, pc_ln_b, pc_w2, pc_b2,
           ks_w1, ks_b1, ks_ln_g, ks_ln_b, ks_w2, ks_b2):
  srcs = jnp.stack([edge_index_0[0], edge_index_1[0], edge_index_2[0]])
  dsts = jnp.stack([edge_index_0[1], edge_index_1[1], edge_index_2[1]])
  wl0 = jnp.stack([wl0_0, wl0_1, wl0_2])
  bl0 = jnp.stack([bl0_0, bl0_1, bl0_2])
  wr0 = jnp.stack([wr0_0, wr0_1, wr0_2])
  wl1 = jnp.stack([wl1_0, wl1_1, wl1_2])
  bl1 = jnp.stack([bl1_0, bl1_1, bl1_2])
  wr1 = jnp.stack([wr1_0, wr1_1, wr1_2])
  zeros = jnp.zeros((N, HALF), _F32)

  z0 = _k1(x_note, wl0, D_IN)
  agg0, cnt = _make_agg(True)(z0, srcs, dsts, zeros)
  x1, z1 = _k2(agg0, cnt, x_note, wr0, bl0, wl1)
  agg1 = _make_agg(False)(z1, srcs, dsts, zeros)[0]
  y, ps, psq = _k3(agg1, cnt, x1, wr1, bl1, w_lin, b_lin[None, :])

  pad = lambda a, r, c: jnp.pad(a, ((0, r - a.shape[0]), (0, c - a.shape[1])))
  pad1 = lambda a, c: jnp.pad(a, ((0, c - a.shape[0],)))[None, :]
  head_args = (
      bn_gamma[None, :], bn_beta[None, :],
      pc_w1, pc_b1[None, :], pc_ln_g[None, :], pc_ln_b[None, :],
      pad(pc_w2, 128, 128), pad1(pc_b2, 128),
      ks_w1[:ENC], pad(ks_w1[ENC:], 128, 128), ks_b1[None, :],
      ks_ln_g[None, :], ks_ln_b[None, :],
      pad(ks_w2, 128, 128), pad1(ks_b2, 128),
  )
  pc_p, ks_p = _k4(y, ps, psq, head_args)
  return pc_p[:, :PC], ks_p[:, :KS]


# speedup vs baseline: 2.9170x; 2.9170x over previous
import jax
import jax.numpy as jnp
from jax import lax
from jax.experimental import pallas as pl
from jax.experimental.pallas import tpu as pltpu
from jax.experimental.pallas import tpu_sc as plsc

N = 10000
E = 320000
D_IN = 128
H = 256
ENC = 256
PC = 35
KS = 15
NET = 3
HALF = 128
NHALF = 5000
NPAD = 5008

BLK = 256
NB = (N + BLK - 1) // BLK

SC_NS = 16
CHUNK = 512
NCHUNK = E // CHUNK

_F32 = jnp.float32


def _make_agg(layer0):
  mesh = plsc.VectorSubcoreMesh(core_axis_name="c", subcore_axis_name="s")
  if layer0:
    out_type = (jax.ShapeDtypeStruct((2, NET, NHALF, HALF), _F32),
                jax.ShapeDtypeStruct((NET, SC_NS, 1, N), _F32))
  else:
    out_type = (jax.ShapeDtypeStruct((NET, 2, N, HALF), _F32),)
  scratch = [
      pltpu.VMEM((CHUNK,), jnp.int32),
      pltpu.VMEM((CHUNK,), jnp.int32),
      pltpu.VMEM((CHUNK,), jnp.int32),
      pltpu.VMEM((CHUNK, HALF), _F32),
      pltpu.VMEM_SHARED((NPAD, HALF), _F32),
      pltpu.SemaphoreType.DMA,
  ]
  if layer0:
    scratch.append(pltpu.VMEM((1, N), _F32))

  def body(x_ref, src0, src1, src2, dst0, dst1, dst2, zero_ref,
           agg_ref, *rest):
    if layer0:
      cnt_ref, idx_s, idx_d, idx_t, rows, acc, sem, cnt_acc = rest
    else:
      idx_s, idx_d, idx_t, rows, acc, sem = rest
    src_list = [src0, src1, src2]
    dst_list = [dst0, dst1, dst2]
    c = lax.axis_index("c")
    s = lax.axis_index("s")
    lanes = lax.iota(jnp.int32, 16)
    ones16 = jnp.ones((16,), _F32)

    def one_pass(e, lo, do_cnt, gather_src):
      @pl.when(s == 0)
      def _zero_acc():
        pltpu.sync_copy(zero_ref, acc)
      plsc.subcore_barrier()

      def chunk_body(i, carry):
        ci = i * SC_NS + s

        @pl.when(ci < NCHUNK)
        def _do_chunk():
          base = ci * CHUNK
          pltpu.sync_copy(src_list[e].at[pl.ds(base, CHUNK)], idx_s)
          pltpu.sync_copy(dst_list[e].at[pl.ds(base, CHUNK)], idx_d)
          for g in range(CHUNK // 16):
            d16 = idx_d[pl.ds(g * 16, 16)]
            loc = d16 - lo
            ok = (loc >= 0) & (loc < NHALF)
            idx_t[pl.ds(g * 16, 16)] = jnp.where(ok, loc, NHALF)
          pltpu.async_copy(gather_src.at[idx_s], rows, sem).wait()
          pltpu.sync_copy(rows, acc.at[idx_t], add=True)
          if do_cnt:
            @pl.when(c == 0)
            def _count():
              for g in range(CHUNK // 16):
                d16 = idx_d[pl.ds(g * 16, 16)]
                for l in range(16):
                  plsc.addupdate_scatter(
                      cnt_acc.at[0], [d16], ones16, mask=lanes == l)
        return carry

      lax.fori_loop(0, (NCHUNK + SC_NS - 1) // SC_NS, chunk_body, 0)
      if do_cnt:
        @pl.when(c == 0)
        def _write_cnt():
          pltpu.sync_copy(cnt_acc, cnt_ref.at[e].at[s])
      plsc.subcore_barrier()

    if layer0:
      lo = c * NHALF
      for e in range(NET):
        @pl.when(c == 0)
        def _zero_cnt():
          def zb(i, carry):
            cnt_acc[0, pl.ds(i * 16, 16)] = jnp.zeros((16,), _F32)
            return carry
          lax.fori_loop(0, N // 16, zb, 0)
        one_pass(e, lo, True, x_ref)
        @pl.when(s == 0)
        def _writeout():
          pltpu.sync_copy(acc.at[pl.ds(0, NHALF)], agg_ref.at[c].at[e])
        plsc.subcore_barrier()
    else:
      for e in range(NET):
        for p in range(2):
          lo = p * NHALF
          one_pass(e, lo, False, x_ref.at[c])
          @pl.when(s == 0)
          def _writeout():
            pltpu.sync_copy(acc.at[pl.ds(0, NHALF)],
                            agg_ref.at[e].at[c].at[pl.ds(lo, NHALF)])
          plsc.subcore_barrier()

  return pl.kernel(body, out_type=out_type, mesh=mesh,
                   scratch_types=tuple(scratch),
                   compiler_params=pltpu.CompilerParams(
                       needs_layout_passes=False))


def _recip_cnt(cnt_ref):
  cnt = jnp.sum(cnt_ref[...], axis=(1, 2))
  return 1.0 / jnp.maximum(cnt, 1.0)


def _k2_body(agg_ref, cnt_ref, x_ref, wl_ref, bl_ref, wr_ref, x1_ref):
  recip = _recip_cnt(cnt_ref)
  x = x_ref[...]
  wr_s = wr_ref[0] + wr_ref[1] + wr_ref[2]
  acc = jnp.dot(x, wr_s, preferred_element_type=_F32, precision=lax.Precision.HIGHEST)
  for e in range(NET):
    mean_e = agg_ref[e] * recip[e][:, None]
    acc = acc + jnp.dot(mean_e, wl_ref[e], preferred_element_type=_F32, precision=lax.Precision.HIGHEST) \
        + bl_ref[e][None, :]
  x1 = jnp.maximum(acc, 0.0)
  x1_ref[0] = x1[:, :HALF]
  x1_ref[1] = x1[:, HALF:]


def _k2(agg0, cnt, x, wl_stack, bl_stack, wr_stack):
  return pl.pallas_call(
      _k2_body,
      grid=(NB,),
      in_specs=[
          pl.BlockSpec((NET, BLK, HALF), lambda i: (0, i, 0)),
          pl.BlockSpec((NET, SC_NS, 1, BLK), lambda i: (0, 0, 0, i)),
          pl.BlockSpec((BLK, D_IN), lambda i: (i, 0)),
          pl.BlockSpec((NET, D_IN, H), lambda i: (0, 0, 0)),
          pl.BlockSpec((NET, H), lambda i: (0, 0)),
          pl.BlockSpec((NET, D_IN, H), lambda i: (0, 0, 0)),
      ],
      out_specs=pl.BlockSpec((2, BLK, HALF), lambda i: (0, i, 0)),
      out_shape=jax.ShapeDtypeStruct((2, N, HALF), _F32),
  )(agg0, cnt, x, wl_stack, bl_stack, wr_stack)


def _k3_body(agg_ref, cnt_ref, x1_ref, wl_ref, bl_ref, wr_ref,
             wlin_ref, blin_ref, y_ref, ps_ref, psq_ref):
  recip = _recip_cnt(cnt_ref)
  x1 = jnp.concatenate([x1_ref[0], x1_ref[1]], axis=1)
  wr_s = wr_ref[0] + wr_ref[1] + wr_ref[2]
  acc = jnp.dot(x1, wr_s, preferred_element_type=_F32, precision=lax.Precision.HIGHEST)
  for e in range(NET):
    agg_e = jnp.concatenate([agg_ref[e, 0], agg_ref[e, 1]], axis=1)
    mean_e = agg_e * recip[e][:, None]
    acc = acc + jnp.dot(mean_e, wl_ref[e], preferred_element_type=_F32, precision=lax.Precision.HIGHEST) \
        + bl_ref[e][None, :]
  x2 = jnp.maximum(acc, 0.0)
  y = jnp.dot(x2, wlin_ref[...], preferred_element_type=_F32, precision=lax.Precision.HIGHEST) + blin_ref[0]
  y_ref[...] = y
  nb = pl.program_id(0)
  row = nb * BLK + lax.broadcasted_iota(jnp.int32, (BLK, 1), 0)
  ym = jnp.where(row < N, y, 0.0)
  ps_ref[0, 0] = jnp.sum(ym, axis=0)
  psq_ref[0, 0] = jnp.sum(ym * ym, axis=0)


def _k3(agg1, cnt, x1b, wl1_stack, bl1_stack, wr1_stack, w_lin, b_lin2):
  return pl.pallas_call(
      _k3_body,
      grid=(NB,),
      in_specs=[
          pl.BlockSpec((NET, 2, BLK, HALF), lambda i: (0, 0, i, 0)),
          pl.BlockSpec((NET, SC_NS, 1, BLK), lambda i: (0, 0, 0, i)),
          pl.BlockSpec((2, BLK, HALF), lambda i: (0, i, 0)),
          pl.BlockSpec((NET, H, H), lambda i: (0, 0, 0)),
          pl.BlockSpec((NET, H), lambda i: (0, 0)),
          pl.BlockSpec((NET, H, H), lambda i: (0, 0, 0)),
          pl.BlockSpec((H, ENC), lambda i: (0, 0)),
          pl.BlockSpec((1, ENC), lambda i: (0, 0)),
      ],
      out_specs=[
          pl.BlockSpec((BLK, ENC), lambda i: (i, 0)),
          pl.BlockSpec((1, 1, ENC), lambda i: (i, 0, 0)),
          pl.BlockSpec((1, 1, ENC), lambda i: (i, 0, 0)),
      ],
      out_shape=[
          jax.ShapeDtypeStruct((N, ENC), _F32),
          jax.ShapeDtypeStruct((NB, 1, ENC), _F32),
          jax.ShapeDtypeStruct((NB, 1, ENC), _F32),
      ],
  )(agg1, cnt, x1b, wl1_stack, bl1_stack, wr1_stack, w_lin, b_lin2)


def _layer_norm_rows(h, g, b):
  hm = jnp.mean(h, axis=1, keepdims=True)
  hv = jnp.mean(h * h, axis=1, keepdims=True) - hm * hm
  return (h - hm) * lax.rsqrt(hv + 1e-5) * g[None, :] + b[None, :]


def _k4_body(y_ref, ps_ref, psq_ref, bng_ref, bnb_ref,
             pw1_ref, pb1_ref, plg_ref, plb_ref, pw2_ref, pb2_ref,
             kw1x_ref, kw1p_ref, kb1_ref, klg_ref, klb_ref,
             kw2_ref, kb2_ref, pc_ref, ks_ref):
  tot = jnp.sum(ps_ref[...], axis=(0, 1))
  tot2 = jnp.sum(psq_ref[...], axis=(0, 1))
  mu = tot / N
  var = tot2 / N - mu * mu
  inv = lax.rsqrt(var + 1e-5)
  y = y_ref[...]
  xb = (y - mu[None, :]) * (inv * bng_ref[0])[None, :] + bnb_ref[0][None, :]
  h = jnp.maximum(jnp.dot(xb, pw1_ref[...], preferred_element_type=_F32, precision=lax.Precision.HIGHEST)
                  + pb1_ref[0][None, :], 0.0)
  h = _layer_norm_rows(h, plg_ref[0], plb_ref[0])
  pc = jnp.dot(h, pw2_ref[...], preferred_element_type=_F32, precision=lax.Precision.HIGHEST) + pb2_ref[0][None, :]
  pc_ref[...] = pc
  h2 = jnp.maximum(jnp.dot(xb, kw1x_ref[...], preferred_element_type=_F32, precision=lax.Precision.HIGHEST)
                   + jnp.dot(pc, kw1p_ref[...], preferred_element_type=_F32, precision=lax.Precision.HIGHEST)
                   + kb1_ref[0][None, :], 0.0)
  h2 = _layer_norm_rows(h2, klg_ref[0], klb_ref[0])
  ks_ref[...] = jnp.dot(h2, kw2_ref[...], preferred_element_type=_F32, precision=lax.Precision.HIGHEST) \
      + kb2_ref[0][None, :]


def _k4(y, ps, psq, args):
  full2 = lambda a: pl.BlockSpec(a.shape, lambda i: tuple(0 for _ in a.shape))
  return pl.pallas_call(
      _k4_body,
      grid=(NB,),
      in_specs=[
          pl.BlockSpec((BLK, ENC), lambda i: (i, 0)),
          pl.BlockSpec((NB, 1, ENC), lambda i: (0, 0, 0)),
          pl.BlockSpec((NB, 1, ENC), lambda i: (0, 0, 0)),
      ] + [full2(a) for a in args],
      out_specs=[
          pl.BlockSpec((BLK, 128), lambda i: (i, 0)),
          pl.BlockSpec((BLK, 128), lambda i: (i, 0)),
      ],
      out_shape=[
          jax.ShapeDtypeStruct((N, 128), _F32),
          jax.ShapeDtypeStruct((N, 128), _F32),
      ],
  )(y, ps, psq, *args)


def kernel(x_note, edge_index_0, edge_index_1, edge_index_2,
           wl0_0, bl0_0, wr0_0, wl0_1, bl0_1, wr0_1, wl0_2, bl0_2, wr0_2,
           wl1_0, bl1_0, wr1_0, wl1_1, bl1_1, wr1_1, wl1_2, bl1_2, wr1_2,
           w_lin, b_lin, bn_gamma, bn_beta,
           pc_w1, pc_b1, pc_ln_g, pc_ln_b, pc_w2, pc_b2,
           ks_w1, ks_b1, ks_ln_g, ks_ln_b, ks_w2, ks_b2):
  edges = (edge_index_0[0], edge_index_1[0], edge_index_2[0],
           edge_index_0[1], edge_index_1[1], edge_index_2[1])
  wl0 = jnp.stack([wl0_0, wl0_1, wl0_2])
  bl0 = jnp.stack([bl0_0, bl0_1, bl0_2])
  wr0 = jnp.stack([wr0_0, wr0_1, wr0_2])
  wl1 = jnp.stack([wl1_0, wl1_1, wl1_2])
  bl1 = jnp.stack([bl1_0, bl1_1, bl1_2])
  wr1 = jnp.stack([wr1_0, wr1_1, wr1_2])
  zeros = jnp.zeros((NPAD, HALF), _F32)

  agg0, cnt = _make_agg(True)(x_note, *edges, zeros)
  agg0 = agg0.transpose(1, 0, 2, 3).reshape(NET, N, HALF)
  x1b = _k2(agg0, cnt, x_note, wl0, bl0, wr0)
  agg1 = _make_agg(False)(x1b, *edges, zeros)[0]
  y, ps, psq = _k3(agg1, cnt, x1b, wl1, bl1, wr1, w_lin, b_lin[None, :])

  pad = lambda a, r, c: jnp.pad(a, ((0, r - a.shape[0]), (0, c - a.shape[1])))
  pad1 = lambda a, c: jnp.pad(a, ((0, c - a.shape[0],)))[None, :]
  head_args = (
      bn_gamma[None, :], bn_beta[None, :],
      pc_w1, pc_b1[None, :], pc_ln_g[None, :], pc_ln_b[None, :],
      pad(pc_w2, 128, 128), pad1(pc_b2, 128),
      ks_w1[:ENC], pad(ks_w1[ENC:], 128, 128), ks_b1[None, :],
      ks_ln_g[None, :], ks_ln_b[None, :],
      pad(ks_w2, 128, 128), pad1(ks_b2, 128),
  )
  pc_p, ks_p = _k4(y, ps, psq, head_args)
  return pc_p[:, :PC], ks_p[:, :KS]


# speedup vs baseline: 5.1136x; 1.7530x over previous
import jax
import jax.numpy as jnp
from jax import lax
from jax.experimental import pallas as pl
from jax.experimental.pallas import tpu as pltpu
from jax.experimental.pallas import tpu_sc as plsc

N = 10000
E = 320000
D_IN = 128
H = 256
ENC = 256
PC = 35
KS = 15
NET = 3
HALF = 128
NPAD = 10016

BLK = 256
NB = (N + BLK - 1) // BLK

SC_NS = 16
CHUNK = 256
NCHUNK = E // CHUNK

_F32 = jnp.float32


def _make_agg(layer0):
  mesh = plsc.VectorSubcoreMesh(core_axis_name="c", subcore_axis_name="s")
  if layer0:
    out_type = (jax.ShapeDtypeStruct((2, NET, N, HALF), _F32),
                jax.ShapeDtypeStruct((NET, SC_NS, 2, 1, N), _F32))
  else:
    out_type = (jax.ShapeDtypeStruct((NET, 2, N, HALF), _F32),)
  scratch = [
      pltpu.VMEM((CHUNK,), jnp.int32),
      pltpu.VMEM((CHUNK,), jnp.int32),
      pltpu.VMEM((CHUNK, HALF), _F32),
      pltpu.VMEM_SHARED((NPAD, HALF), _F32),
      pltpu.SemaphoreType.DMA,
  ]
  if layer0:
    scratch.append(pltpu.VMEM((1, N), _F32))

  def body(x_ref, src0, src1, src2, dst0, dst1, dst2, zero_ref,
           agg_ref, *rest):
    if layer0:
      cnt_ref, idx_s, idx_d, rows, acc, sem, cnt_acc = rest
    else:
      idx_s, idx_d, rows, acc, sem = rest
    src_list = [src0, src1, src2]
    dst_list = [dst0, dst1, dst2]
    c = lax.axis_index("c")
    s = lax.axis_index("s")
    lanes = lax.iota(jnp.int32, 16)
    ones16 = jnp.ones((16,), _F32)

    if layer0:
      ci_lo = c * (NCHUNK // 2)
      ci_hi = ci_lo + NCHUNK // 2
      niter = (NCHUNK // 2 + SC_NS - 1) // SC_NS
    else:
      ci_lo = 0
      ci_hi = NCHUNK
      niter = (NCHUNK + SC_NS - 1) // SC_NS

    for e in range(NET):
      if layer0:
        def _zero_cnt(i, carry):
          cnt_acc[0, pl.ds(i * 16, 16)] = jnp.zeros((16,), _F32)
          return carry
        lax.fori_loop(0, N // 16, _zero_cnt, 0)

      @pl.when(s == 0)
      def _zero_acc():
        pltpu.sync_copy(zero_ref, acc)
      plsc.subcore_barrier()

      def chunk_body(i, carry, e=e):
        ci = ci_lo + i * SC_NS + s

        @pl.when(ci < ci_hi)
        def _do_chunk():
          base = ci * CHUNK
          pltpu.sync_copy(src_list[e].at[pl.ds(base, CHUNK)], idx_s)
          pltpu.sync_copy(dst_list[e].at[pl.ds(base, CHUNK)], idx_d)
          gsrc = x_ref if layer0 else x_ref.at[c]
          pltpu.async_copy(gsrc.at[idx_s], rows, sem).wait()
          pltpu.sync_copy(rows, acc.at[idx_d], add=True)
          if layer0:
            for g in range(CHUNK // 16):
              d16 = idx_d[pl.ds(g * 16, 16)]
              for l in range(16):
                plsc.addupdate_scatter(
                    cnt_acc.at[0], [d16], ones16, mask=lanes == l)
        return carry

      lax.fori_loop(0, niter, chunk_body, 0)
      if layer0:
        pltpu.sync_copy(cnt_acc, cnt_ref.at[e].at[s].at[c])
      plsc.subcore_barrier()

      @pl.when(s == 0)
      def _writeout():
        dst = agg_ref.at[c].at[e] if layer0 else agg_ref.at[e].at[c]
        pltpu.sync_copy(acc.at[pl.ds(0, N)], dst)
      plsc.subcore_barrier()

  return pl.kernel(body, out_type=out_type, mesh=mesh,
                   scratch_types=tuple(scratch),
                   compiler_params=pltpu.CompilerParams(
                       needs_layout_passes=False))


def _recip_cnt(cnt_ref):
  cnt = jnp.sum(cnt_ref[...], axis=(1, 2, 3))
  return 1.0 / jnp.maximum(cnt, 1.0)


def _k2_body(agg_ref, cnt_ref, x_ref, wl_ref, bl_ref, wr_ref, x1_ref):
  recip = _recip_cnt(cnt_ref)
  x = x_ref[...]
  wr_s = wr_ref[0] + wr_ref[1] + wr_ref[2]
  acc = jnp.dot(x, wr_s, preferred_element_type=_F32, precision=lax.Precision.HIGHEST)
  for e in range(NET):
    mean_e = (agg_ref[0, e] + agg_ref[1, e]) * recip[e][:, None]
    acc = acc + jnp.dot(mean_e, wl_ref[e], preferred_element_type=_F32, precision=lax.Precision.HIGHEST) \
        + bl_ref[e][None, :]
  x1 = jnp.maximum(acc, 0.0)
  x1_ref[0] = x1[:, :HALF]
  x1_ref[1] = x1[:, HALF:]


def _k2(agg0, cnt, x, wl_stack, bl_stack, wr_stack):
  return pl.pallas_call(
      _k2_body,
      grid=(NB,),
      in_specs=[
          pl.BlockSpec((2, NET, BLK, HALF), lambda i: (0, 0, i, 0)),
          pl.BlockSpec((NET, SC_NS, 2, 1, BLK), lambda i: (0, 0, 0, 0, i)),
          pl.BlockSpec((BLK, D_IN), lambda i: (i, 0)),
          pl.BlockSpec((NET, D_IN, H), lambda i: (0, 0, 0)),
          pl.BlockSpec((NET, H), lambda i: (0, 0)),
          pl.BlockSpec((NET, D_IN, H), lambda i: (0, 0, 0)),
      ],
      out_specs=pl.BlockSpec((2, BLK, HALF), lambda i: (0, i, 0)),
      out_shape=jax.ShapeDtypeStruct((2, N, HALF), _F32),
  )(agg0, cnt, x, wl_stack, bl_stack, wr_stack)


def _k3_body(agg_ref, cnt_ref, x1_ref, wl_ref, bl_ref, wr_ref,
             wlin_ref, blin_ref, y_ref, ps_ref, psq_ref):
  recip = _recip_cnt(cnt_ref)
  x1 = jnp.concatenate([x1_ref[0], x1_ref[1]], axis=1)
  wr_s = wr_ref[0] + wr_ref[1] + wr_ref[2]
  acc = jnp.dot(x1, wr_s, preferred_element_type=_F32, precision=lax.Precision.HIGHEST)
  for e in range(NET):
    agg_e = jnp.concatenate([agg_ref[e, 0], agg_ref[e, 1]], axis=1)
    mean_e = agg_e * recip[e][:, None]
    acc = acc + jnp.dot(mean_e, wl_ref[e], preferred_element_type=_F32, precision=lax.Precision.HIGHEST) \
        + bl_ref[e][None, :]
  x2 = jnp.maximum(acc, 0.0)
  y = jnp.dot(x2, wlin_ref[...], preferred_element_type=_F32, precision=lax.Precision.HIGHEST) + blin_ref[0]
  y_ref[...] = y
  nb = pl.program_id(0)
  row = nb * BLK + lax.broadcasted_iota(jnp.int32, (BLK, 1), 0)
  ym = jnp.where(row < N, y, 0.0)
  ps_ref[0, 0] = jnp.sum(ym, axis=0)
  psq_ref[0, 0] = jnp.sum(ym * ym, axis=0)


def _k3(agg1, cnt, x1b, wl1_stack, bl1_stack, wr1_stack, w_lin, b_lin2):
  return pl.pallas_call(
      _k3_body,
      grid=(NB,),
      in_specs=[
          pl.BlockSpec((NET, 2, BLK, HALF), lambda i: (0, 0, i, 0)),
          pl.BlockSpec((NET, SC_NS, 2, 1, BLK), lambda i: (0, 0, 0, 0, i)),
          pl.BlockSpec((2, BLK, HALF), lambda i: (0, i, 0)),
          pl.BlockSpec((NET, H, H), lambda i: (0, 0, 0)),
          pl.BlockSpec((NET, H), lambda i: (0, 0)),
          pl.BlockSpec((NET, H, H), lambda i: (0, 0, 0)),
          pl.BlockSpec((H, ENC), lambda i: (0, 0)),
          pl.BlockSpec((1, ENC), lambda i: (0, 0)),
      ],
      out_specs=[
          pl.BlockSpec((BLK, ENC), lambda i: (i, 0)),
          pl.BlockSpec((1, 1, ENC), lambda i: (i, 0, 0)),
          pl.BlockSpec((1, 1, ENC), lambda i: (i, 0, 0)),
      ],
      out_shape=[
          jax.ShapeDtypeStruct((N, ENC), _F32),
          jax.ShapeDtypeStruct((NB, 1, ENC), _F32),
          jax.ShapeDtypeStruct((NB, 1, ENC), _F32),
      ],
  )(agg1, cnt, x1b, wl1_stack, bl1_stack, wr1_stack, w_lin, b_lin2)


def _layer_norm_rows(h, g, b):
  hm = jnp.mean(h, axis=1, keepdims=True)
  hv = jnp.mean(h * h, axis=1, keepdims=True) - hm * hm
  return (h - hm) * lax.rsqrt(hv + 1e-5) * g[None, :] + b[None, :]


def _k4_body(y_ref, ps_ref, psq_ref, bng_ref, bnb_ref,
             pw1_ref, pb1_ref, plg_ref, plb_ref, pw2_ref, pb2_ref,
             kw1x_ref, kw1p_ref, kb1_ref, klg_ref, klb_ref,
             kw2_ref, kb2_ref, pc_ref, ks_ref):
  tot = jnp.sum(ps_ref[...], axis=(0, 1))
  tot2 = jnp.sum(psq_ref[...], axis=(0, 1))
  mu = tot / N
  var = tot2 / N - mu * mu
  inv = lax.rsqrt(var + 1e-5)
  y = y_ref[...]
  xb = (y - mu[None, :]) * (inv * bng_ref[0])[None, :] + bnb_ref[0][None, :]
  h = jnp.maximum(jnp.dot(xb, pw1_ref[...], preferred_element_type=_F32, precision=lax.Precision.HIGHEST)
                  + pb1_ref[0][None, :], 0.0)
  h = _layer_norm_rows(h, plg_ref[0], plb_ref[0])
  pc = jnp.dot(h, pw2_ref[...], preferred_element_type=_F32, precision=lax.Precision.HIGHEST) + pb2_ref[0][None, :]
  pc_ref[...] = pc
  h2 = jnp.maximum(jnp.dot(xb, kw1x_ref[...], preferred_element_type=_F32, precision=lax.Precision.HIGHEST)
                   + jnp.dot(pc, kw1p_ref[...], preferred_element_type=_F32, precision=lax.Precision.HIGHEST)
                   + kb1_ref[0][None, :], 0.0)
  h2 = _layer_norm_rows(h2, klg_ref[0], klb_ref[0])
  ks_ref[...] = jnp.dot(h2, kw2_ref[...], preferred_element_type=_F32, precision=lax.Precision.HIGHEST) \
      + kb2_ref[0][None, :]


def _k4(y, ps, psq, args):
  full2 = lambda a: pl.BlockSpec(a.shape, lambda i: tuple(0 for _ in a.shape))
  return pl.pallas_call(
      _k4_body,
      grid=(NB,),
      in_specs=[
          pl.BlockSpec((BLK, ENC), lambda i: (i, 0)),
          pl.BlockSpec((NB, 1, ENC), lambda i: (0, 0, 0)),
          pl.BlockSpec((NB, 1, ENC), lambda i: (0, 0, 0)),
      ] + [full2(a) for a in args],
      out_specs=[
          pl.BlockSpec((BLK, 128), lambda i: (i, 0)),
          pl.BlockSpec((BLK, 128), lambda i: (i, 0)),
      ],
      out_shape=[
          jax.ShapeDtypeStruct((N, 128), _F32),
          jax.ShapeDtypeStruct((N, 128), _F32),
      ],
  )(y, ps, psq, *args)


def kernel(x_note, edge_index_0, edge_index_1, edge_index_2,
           wl0_0, bl0_0, wr0_0, wl0_1, bl0_1, wr0_1, wl0_2, bl0_2, wr0_2,
           wl1_0, bl1_0, wr1_0, wl1_1, bl1_1, wr1_1, wl1_2, bl1_2, wr1_2,
           w_lin, b_lin, bn_gamma, bn_beta,
           pc_w1, pc_b1, pc_ln_g, pc_ln_b, pc_w2, pc_b2,
           ks_w1, ks_b1, ks_ln_g, ks_ln_b, ks_w2, ks_b2):
  edges = (edge_index_0[0], edge_index_1[0], edge_index_2[0],
           edge_index_0[1], edge_index_1[1], edge_index_2[1])
  wl0 = jnp.stack([wl0_0, wl0_1, wl0_2])
  bl0 = jnp.stack([bl0_0, bl0_1, bl0_2])
  wr0 = jnp.stack([wr0_0, wr0_1, wr0_2])
  wl1 = jnp.stack([wl1_0, wl1_1, wl1_2])
  bl1 = jnp.stack([bl1_0, bl1_1, bl1_2])
  wr1 = jnp.stack([wr1_0, wr1_1, wr1_2])
  zeros = jnp.zeros((NPAD, HALF), _F32)

  agg0, cnt = _make_agg(True)(x_note, *edges, zeros)
  x1b = _k2(agg0, cnt, x_note, wl0, bl0, wr0)
  agg1 = _make_agg(False)(x1b, *edges, zeros)[0]
  y, ps, psq = _k3(agg1, cnt, x1b, wl1, bl1, wr1, w_lin, b_lin[None, :])

  pad = lambda a, r, c: jnp.pad(a, ((0, r - a.shape[0]), (0, c - a.shape[1])))
  pad1 = lambda a, c: jnp.pad(a, ((0, c - a.shape[0],)))[None, :]
  head_args = (
      bn_gamma[None, :], bn_beta[None, :],
      pc_w1, pc_b1[None, :], pc_ln_g[None, :], pc_ln_b[None, :],
      pad(pc_w2, 128, 128), pad1(pc_b2, 128),
      ks_w1[:ENC], pad(ks_w1[ENC:], 128, 128), ks_b1[None, :],
      ks_ln_g[None, :], ks_ln_b[None, :],
      pad(ks_w2, 128, 128), pad1(ks_b2, 128),
  )
  pc_p, ks_p = _k4(y, ps, psq, head_args)
  return pc_p[:, :PC], ks_p[:, :KS]


# speedup vs baseline: 5.6572x; 1.1063x over previous
import jax
import jax.numpy as jnp
from jax import lax
from jax.experimental import pallas as pl
from jax.experimental.pallas import tpu as pltpu
from jax.experimental.pallas import tpu_sc as plsc

N = 10000
E = 320000
D_IN = 128
H = 256
ENC = 256
PC = 35
KS = 15
NET = 3
HALF = 128
NPAD = 10016

BLK = 256
NB = (N + BLK - 1) // BLK

SC_NS = 16
CHUNK = 128
NCHUNK = E // CHUNK

_F32 = jnp.float32


def _make_agg(layer0):
  mesh = plsc.VectorSubcoreMesh(core_axis_name="c", subcore_axis_name="s")
  if layer0:
    out_type = (jax.ShapeDtypeStruct((2, NET, N, HALF), _F32),
                jax.ShapeDtypeStruct((NET, SC_NS, 2, 1, N), _F32))
  else:
    out_type = (jax.ShapeDtypeStruct((NET, 2, N, HALF), _F32),)
  scratch = [
      pltpu.VMEM((2, CHUNK), jnp.int32),
      pltpu.VMEM((2, CHUNK), jnp.int32),
      pltpu.VMEM((2, CHUNK, HALF), _F32),
      pltpu.VMEM_SHARED((NPAD, HALF), _F32),
      pltpu.SemaphoreType.DMA,
      pltpu.SemaphoreType.DMA,
      pltpu.SemaphoreType.DMA,
  ]
  if layer0:
    scratch.append(pltpu.VMEM((1, N), _F32))

  def body(x_ref, src0, src1, src2, dst0, dst1, dst2, zero_ref,
           agg_ref, *rest):
    if layer0:
      cnt_ref, idx_s, idx_d, rows, acc, isem, gsem, ssem, cnt_acc = rest
    else:
      idx_s, idx_d, rows, acc, isem, gsem, ssem = rest
    src_list = [src0, src1, src2]
    dst_list = [dst0, dst1, dst2]
    c = lax.axis_index("c")
    s = lax.axis_index("s")
    lanes = lax.iota(jnp.int32, 16)
    ones16 = jnp.ones((16,), _F32)

    if layer0:
      ci_lo = c * (NCHUNK // 2)
      ci_hi = ci_lo + NCHUNK // 2
      niter = (NCHUNK // 2 + SC_NS - 1) // SC_NS
    else:
      ci_lo = 0
      ci_hi = NCHUNK
      niter = (NCHUNK + SC_NS - 1) // SC_NS

    for e in range(NET):
      if layer0:
        def _zero_cnt(i, carry):
          cnt_acc[0, pl.ds(i * 16, 16)] = jnp.zeros((16,), _F32)
          return carry
        lax.fori_loop(0, N // 16, _zero_cnt, 0)

      @pl.when(s == 0)
      def _zero_acc():
        pltpu.sync_copy(zero_ref, acc)
      plsc.subcore_barrier()

      gsrc = x_ref if layer0 else x_ref.at[c]

      def pair_body(i, carry, e=e):
        cis = [ci_lo + (2 * i + b) * SC_NS + s for b in range(2)]
        for b in range(2):
          @pl.when(cis[b] < ci_hi)
          def _fetch(b=b, ci=cis[b]):
            base = ci * CHUNK
            pltpu.async_copy(src_list[e].at[pl.ds(base, CHUNK)],
                             idx_s.at[b], isem)
            pltpu.async_copy(dst_list[e].at[pl.ds(base, CHUNK)],
                             idx_d.at[b], isem)
        for b in range(2):
          @pl.when(cis[b] < ci_hi)
          def _drain_idx(b=b, ci=cis[b]):
            base = ci * CHUNK
            pltpu.make_async_copy(src_list[e].at[pl.ds(base, CHUNK)],
                                  idx_s.at[b], isem).wait()
            pltpu.make_async_copy(dst_list[e].at[pl.ds(base, CHUNK)],
                                  idx_d.at[b], isem).wait()
        for b in range(2):
          @pl.when(cis[b] < ci_hi)
          def _gather(b=b):
            pltpu.async_copy(gsrc.at[idx_s.at[b]], rows.at[b], gsem)
        for b in range(2):
          @pl.when(cis[b] < ci_hi)
          def _drain_gather(b=b):
            pltpu.make_async_copy(gsrc.at[idx_s.at[b]], rows.at[b],
                                  gsem).wait()
        for b in range(2):
          @pl.when(cis[b] < ci_hi)
          def _scatter(b=b):
            pltpu.async_copy(rows.at[b], acc.at[idx_d.at[b]], ssem,
                             add=True)
            if layer0:
              for g in range(CHUNK // 16):
                d16 = idx_d[b, pl.ds(g * 16, 16)]
                for l in range(16):
                  plsc.addupdate_scatter(
                      cnt_acc.at[0], [d16], ones16, mask=lanes == l)
        for b in range(2):
          @pl.when(cis[b] < ci_hi)
          def _drain(b=b):
            pltpu.make_async_copy(rows.at[b], acc.at[idx_d.at[b]],
                                  ssem).wait()
        return carry

      lax.fori_loop(0, (niter + 1) // 2, pair_body, 0)
      if layer0:
        pltpu.sync_copy(cnt_acc, cnt_ref.at[e].at[s].at[c])
      plsc.subcore_barrier()

      @pl.when(s == 0)
      def _writeout():
        dst = agg_ref.at[c].at[e] if layer0 else agg_ref.at[e].at[c]
        pltpu.sync_copy(acc.at[pl.ds(0, N)], dst)
      plsc.subcore_barrier()

  return pl.kernel(body, out_type=out_type, mesh=mesh,
                   scratch_types=tuple(scratch),
                   compiler_params=pltpu.CompilerParams(
                       needs_layout_passes=False))


def _recip_cnt(cnt_ref):
  cnt = jnp.sum(cnt_ref[...], axis=(1, 2, 3))
  return 1.0 / jnp.maximum(cnt, 1.0)


def _k2_body(agg_ref, cnt_ref, x_ref, wl_ref, bl_ref, wr_ref, x1_ref):
  recip = _recip_cnt(cnt_ref)
  x = x_ref[...]
  wr_s = wr_ref[0] + wr_ref[1] + wr_ref[2]
  acc = jnp.dot(x, wr_s, preferred_element_type=_F32, precision=lax.Precision.HIGHEST)
  for e in range(NET):
    mean_e = (agg_ref[0, e] + agg_ref[1, e]) * recip[e][:, None]
    acc = acc + jnp.dot(mean_e, wl_ref[e], preferred_element_type=_F32, precision=lax.Precision.HIGHEST) \
        + bl_ref[e][None, :]
  x1 = jnp.maximum(acc, 0.0)
  x1_ref[0] = x1[:, :HALF]
  x1_ref[1] = x1[:, HALF:]


def _k2(agg0, cnt, x, wl_stack, bl_stack, wr_stack):
  return pl.pallas_call(
      _k2_body,
      grid=(NB,),
      in_specs=[
          pl.BlockSpec((2, NET, BLK, HALF), lambda i: (0, 0, i, 0)),
          pl.BlockSpec((NET, SC_NS, 2, 1, BLK), lambda i: (0, 0, 0, 0, i)),
          pl.BlockSpec((BLK, D_IN), lambda i: (i, 0)),
          pl.BlockSpec((NET, D_IN, H), lambda i: (0, 0, 0)),
          pl.BlockSpec((NET, H), lambda i: (0, 0)),
          pl.BlockSpec((NET, D_IN, H), lambda i: (0, 0, 0)),
      ],
      out_specs=pl.BlockSpec((2, BLK, HALF), lambda i: (0, i, 0)),
      out_shape=jax.ShapeDtypeStruct((2, N, HALF), _F32),
  )(agg0, cnt, x, wl_stack, bl_stack, wr_stack)


def _k3_body(agg_ref, cnt_ref, x1_ref, wl_ref, bl_ref, wr_ref,
             wlin_ref, blin_ref, y_ref, ps_ref, psq_ref):
  recip = _recip_cnt(cnt_ref)
  x1 = jnp.concatenate([x1_ref[0], x1_ref[1]], axis=1)
  wr_s = wr_ref[0] + wr_ref[1] + wr_ref[2]
  acc = jnp.dot(x1, wr_s, preferred_element_type=_F32, precision=lax.Precision.HIGHEST)
  for e in range(NET):
    agg_e = jnp.concatenate([agg_ref[e, 0], agg_ref[e, 1]], axis=1)
    mean_e = agg_e * recip[e][:, None]
    acc = acc + jnp.dot(mean_e, wl_ref[e], preferred_element_type=_F32, precision=lax.Precision.HIGHEST) \
        + bl_ref[e][None, :]
  x2 = jnp.maximum(acc, 0.0)
  y = jnp.dot(x2, wlin_ref[...], preferred_element_type=_F32, precision=lax.Precision.HIGHEST) + blin_ref[0]
  y_ref[...] = y
  nb = pl.program_id(0)
  row = nb * BLK + lax.broadcasted_iota(jnp.int32, (BLK, 1), 0)
  ym = jnp.where(row < N, y, 0.0)
  ps_ref[0, 0] = jnp.sum(ym, axis=0)
  psq_ref[0, 0] = jnp.sum(ym * ym, axis=0)


def _k3(agg1, cnt, x1b, wl1_stack, bl1_stack, wr1_stack, w_lin, b_lin2):
  return pl.pallas_call(
      _k3_body,
      grid=(NB,),
      in_specs=[
          pl.BlockSpec((NET, 2, BLK, HALF), lambda i: (0, 0, i, 0)),
          pl.BlockSpec((NET, SC_NS, 2, 1, BLK), lambda i: (0, 0, 0, 0, i)),
          pl.BlockSpec((2, BLK, HALF), lambda i: (0, i, 0)),
          pl.BlockSpec((NET, H, H), lambda i: (0, 0, 0)),
          pl.BlockSpec((NET, H), lambda i: (0, 0)),
          pl.BlockSpec((NET, H, H), lambda i: (0, 0, 0)),
          pl.BlockSpec((H, ENC), lambda i: (0, 0)),
          pl.BlockSpec((1, ENC), lambda i: (0, 0)),
      ],
      out_specs=[
          pl.BlockSpec((BLK, ENC), lambda i: (i, 0)),
          pl.BlockSpec((1, 1, ENC), lambda i: (i, 0, 0)),
          pl.BlockSpec((1, 1, ENC), lambda i: (i, 0, 0)),
      ],
      out_shape=[
          jax.ShapeDtypeStruct((N, ENC), _F32),
          jax.ShapeDtypeStruct((NB, 1, ENC), _F32),
          jax.ShapeDtypeStruct((NB, 1, ENC), _F32),
      ],
  )(agg1, cnt, x1b, wl1_stack, bl1_stack, wr1_stack, w_lin, b_lin2)


def _layer_norm_rows(h, g, b):
  hm = jnp.mean(h, axis=1, keepdims=True)
  hv = jnp.mean(h * h, axis=1, keepdims=True) - hm * hm
  return (h - hm) * lax.rsqrt(hv + 1e-5) * g[None, :] + b[None, :]


def _k4_body(y_ref, ps_ref, psq_ref, bng_ref, bnb_ref,
             pw1_ref, pb1_ref, plg_ref, plb_ref, pw2_ref, pb2_ref,
             kw1x_ref, kw1p_ref, kb1_ref, klg_ref, klb_ref,
             kw2_ref, kb2_ref, pc_ref, ks_ref):
  tot = jnp.sum(ps_ref[...], axis=(0, 1))
  tot2 = jnp.sum(psq_ref[...], axis=(0, 1))
  mu = tot / N
  var = tot2 / N - mu * mu
  inv = lax.rsqrt(var + 1e-5)
  y = y_ref[...]
  xb = (y - mu[None, :]) * (inv * bng_ref[0])[None, :] + bnb_ref[0][None, :]
  h = jnp.maximum(jnp.dot(xb, pw1_ref[...], preferred_element_type=_F32, precision=lax.Precision.HIGHEST)
                  + pb1_ref[0][None, :], 0.0)
  h = _layer_norm_rows(h, plg_ref[0], plb_ref[0])
  pc = jnp.dot(h, pw2_ref[...], preferred_element_type=_F32, precision=lax.Precision.HIGHEST) + pb2_ref[0][None, :]
  pc_ref[...] = pc
  h2 = jnp.maximum(jnp.dot(xb, kw1x_ref[...], preferred_element_type=_F32, precision=lax.Precision.HIGHEST)
                   + jnp.dot(pc, kw1p_ref[...], preferred_element_type=_F32, precision=lax.Precision.HIGHEST)
                   + kb1_ref[0][None, :], 0.0)
  h2 = _layer_norm_rows(h2, klg_ref[0], klb_ref[0])
  ks_ref[...] = jnp.dot(h2, kw2_ref[...], preferred_element_type=_F32, precision=lax.Precision.HIGHEST) \
      + kb2_ref[0][None, :]


def _k4(y, ps, psq, args):
  full2 = lambda a: pl.BlockSpec(a.shape, lambda i: tuple(0 for _ in a.shape))
  return pl.pallas_call(
      _k4_body,
      grid=(NB,),
      in_specs=[
          pl.BlockSpec((BLK, ENC), lambda i: (i, 0)),
          pl.BlockSpec((NB, 1, ENC), lambda i: (0, 0, 0)),
          pl.BlockSpec((NB, 1, ENC), lambda i: (0, 0, 0)),
      ] + [full2(a) for a in args],
      out_specs=[
          pl.BlockSpec((BLK, 128), lambda i: (i, 0)),
          pl.BlockSpec((BLK, 128), lambda i: (i, 0)),
      ],
      out_shape=[
          jax.ShapeDtypeStruct((N, 128), _F32),
          jax.ShapeDtypeStruct((N, 128), _F32),
      ],
  )(y, ps, psq, *args)


def kernel(x_note, edge_index_0, edge_index_1, edge_index_2,
           wl0_0, bl0_0, wr0_0, wl0_1, bl0_1, wr0_1, wl0_2, bl0_2, wr0_2,
           wl1_0, bl1_0, wr1_0, wl1_1, bl1_1, wr1_1, wl1_2, bl1_2, wr1_2,
           w_lin, b_lin, bn_gamma, bn_beta,
           pc_w1, pc_b1, pc_ln_g, pc_ln_b, pc_w2, pc_b2,
           ks_w1, ks_b1, ks_ln_g, ks_ln_b, ks_w2, ks_b2):
  edges = (edge_index_0[0], edge_index_1[0], edge_index_2[0],
           edge_index_0[1], edge_index_1[1], edge_index_2[1])
  wl0 = jnp.stack([wl0_0, wl0_1, wl0_2])
  bl0 = jnp.stack([bl0_0, bl0_1, bl0_2])
  wr0 = jnp.stack([wr0_0, wr0_1, wr0_2])
  wl1 = jnp.stack([wl1_0, wl1_1, wl1_2])
  bl1 = jnp.stack([bl1_0, bl1_1, bl1_2])
  wr1 = jnp.stack([wr1_0, wr1_1, wr1_2])
  zeros = jnp.zeros((NPAD, HALF), _F32)

  agg0, cnt = _make_agg(True)(x_note, *edges, zeros)
  x1b = _k2(agg0, cnt, x_note, wl0, bl0, wr0)
  agg1 = _make_agg(False)(x1b, *edges, zeros)[0]
  y, ps, psq = _k3(agg1, cnt, x1b, wl1, bl1, wr1, w_lin, b_lin[None, :])

  pad = lambda a, r, c: jnp.pad(a, ((0, r - a.shape[0]), (0, c - a.shape[1])))
  pad1 = lambda a, c: jnp.pad(a, ((0, c - a.shape[0],)))[None, :]
  head_args = (
      bn_gamma[None, :], bn_beta[None, :],
      pc_w1, pc_b1[None, :], pc_ln_g[None, :], pc_ln_b[None, :],
      pad(pc_w2, 128, 128), pad1(pc_b2, 128),
      ks_w1[:ENC], pad(ks_w1[ENC:], 128, 128), ks_b1[None, :],
      ks_ln_g[None, :], ks_ln_b[None, :],
      pad(ks_w2, 128, 128), pad1(ks_b2, 128),
  )
  pc_p, ks_p = _k4(y, ps, psq, head_args)
  return pc_p[:, :PC], ks_p[:, :KS]


# speedup vs baseline: 5.6633x; 1.0011x over previous
import jax
import jax.numpy as jnp
from jax import lax
from jax.experimental import pallas as pl
from jax.experimental.pallas import tpu as pltpu
from jax.experimental.pallas import tpu_sc as plsc

N = 10000
E = 320000
D_IN = 128
H = 256
ENC = 256
PC = 35
KS = 15
NET = 3
HALF = 128
NPAD = 10016

BLK = 256
NB = (N + BLK - 1) // BLK

SC_NS = 16
CHUNK = 128
NCHUNK = E // CHUNK

_F32 = jnp.float32


def _make_agg(layer0):
  mesh = plsc.VectorSubcoreMesh(core_axis_name="c", subcore_axis_name="s")
  if layer0:
    out_type = (jax.ShapeDtypeStruct((2, NET, N, HALF), _F32),
                jax.ShapeDtypeStruct((NET, SC_NS, 2, 1, N), _F32))
  else:
    out_type = (jax.ShapeDtypeStruct((NET, 2, N, HALF), _F32),)
  scratch = [
      pltpu.VMEM((2, CHUNK), jnp.int32),
      pltpu.VMEM((2, CHUNK), jnp.int32),
      pltpu.VMEM((2, CHUNK, HALF), _F32),
      pltpu.VMEM_SHARED((NPAD, HALF), _F32),
      pltpu.SemaphoreType.DMA,
      pltpu.SemaphoreType.DMA,
      pltpu.SemaphoreType.DMA,
  ]
  if layer0:
    scratch.append(pltpu.VMEM((1, N), _F32))

  def body(x_ref, src0, src1, src2, dst0, dst1, dst2, zero_ref,
           agg_ref, *rest):
    if layer0:
      cnt_ref, idx_s, idx_d, rows, acc, isem, gsem, ssem, cnt_acc = rest
    else:
      idx_s, idx_d, rows, acc, isem, gsem, ssem = rest
    src_list = [src0, src1, src2]
    dst_list = [dst0, dst1, dst2]
    c = lax.axis_index("c")
    s = lax.axis_index("s")
    lanes = lax.iota(jnp.int32, 16)
    ones16 = jnp.ones((16,), _F32)

    if layer0:
      ci_lo = c * (NCHUNK // 2)
      ci_hi = ci_lo + NCHUNK // 2
      niter = (NCHUNK // 2 + SC_NS - 1) // SC_NS
    else:
      ci_lo = 0
      ci_hi = NCHUNK
      niter = (NCHUNK + SC_NS - 1) // SC_NS

    for e in range(NET):
      if layer0:
        def _zero_cnt(i, carry):
          cnt_acc[0, pl.ds(i * 16, 16)] = jnp.zeros((16,), _F32)
          return carry
        lax.fori_loop(0, N // 16, _zero_cnt, 0)

      @pl.when(s == 0)
      def _zero_acc():
        pltpu.sync_copy(zero_ref, acc)
      plsc.subcore_barrier()

      gsrc = x_ref if layer0 else x_ref.at[c]

      def pair_body(i, carry, e=e):
        cis = [ci_lo + (2 * i + b) * SC_NS + s for b in range(2)]
        for b in range(2):
          prev = cis[b] - 2 * SC_NS

          @pl.when((i > 0) & (prev < ci_hi))
          def _drain_prev_scatter(b=b):
            pltpu.make_async_copy(rows.at[b], acc.at[idx_d.at[b]],
                                  ssem).wait()
        for b in range(2):
          @pl.when(cis[b] < ci_hi)
          def _fetch(b=b, ci=cis[b]):
            base = ci * CHUNK
            pltpu.async_copy(src_list[e].at[pl.ds(base, CHUNK)],
                             idx_s.at[b], isem)
            pltpu.async_copy(dst_list[e].at[pl.ds(base, CHUNK)],
                             idx_d.at[b], isem)
        for b in range(2):
          @pl.when(cis[b] < ci_hi)
          def _drain_idx(b=b, ci=cis[b]):
            base = ci * CHUNK
            pltpu.make_async_copy(src_list[e].at[pl.ds(base, CHUNK)],
                                  idx_s.at[b], isem).wait()
            pltpu.make_async_copy(dst_list[e].at[pl.ds(base, CHUNK)],
                                  idx_d.at[b], isem).wait()
        for b in range(2):
          @pl.when(cis[b] < ci_hi)
          def _gather(b=b):
            pltpu.async_copy(gsrc.at[idx_s.at[b]], rows.at[b], gsem)
        for b in range(2):
          @pl.when(cis[b] < ci_hi)
          def _drain_gather(b=b):
            pltpu.make_async_copy(gsrc.at[idx_s.at[b]], rows.at[b],
                                  gsem).wait()
        for b in range(2):
          @pl.when(cis[b] < ci_hi)
          def _scatter(b=b):
            pltpu.async_copy(rows.at[b], acc.at[idx_d.at[b]], ssem,
                             add=True)
            if layer0:
              for g in range(CHUNK // 16):
                d16 = idx_d[b, pl.ds(g * 16, 16)]
                for l in range(16):
                  plsc.addupdate_scatter(
                      cnt_acc.at[0], [d16], ones16, mask=lanes == l)
        return carry

      npairs = (niter + 1) // 2
      lax.fori_loop(0, npairs, pair_body, 0)
      for b in range(2):
        lci = ci_lo + (2 * (npairs - 1) + b) * SC_NS + s

        @pl.when(lci < ci_hi)
        def _drain_last(b=b):
          pltpu.make_async_copy(rows.at[b], acc.at[idx_d.at[b]],
                                ssem).wait()
      if layer0:
        pltpu.sync_copy(cnt_acc, cnt_ref.at[e].at[s].at[c])
      plsc.subcore_barrier()

      @pl.when(s == 0)
      def _writeout():
        dst = agg_ref.at[c].at[e] if layer0 else agg_ref.at[e].at[c]
        pltpu.sync_copy(acc.at[pl.ds(0, N)], dst)
      plsc.subcore_barrier()

  return pl.kernel(body, out_type=out_type, mesh=mesh,
                   scratch_types=tuple(scratch),
                   compiler_params=pltpu.CompilerParams(
                       needs_layout_passes=False))


def _recip_cnt(cnt_ref):
  cnt = jnp.sum(cnt_ref[...], axis=(1, 2, 3))
  return 1.0 / jnp.maximum(cnt, 1.0)


def _k2_body(agg_ref, cnt_ref, x_ref, wl_ref, bl_ref, wr_ref, x1_ref):
  recip = _recip_cnt(cnt_ref)
  x = x_ref[...]
  wr_s = wr_ref[0] + wr_ref[1] + wr_ref[2]
  acc = jnp.dot(x, wr_s, preferred_element_type=_F32, precision=lax.Precision.HIGHEST)
  for e in range(NET):
    mean_e = (agg_ref[0, e] + agg_ref[1, e]) * recip[e][:, None]
    acc = acc + jnp.dot(mean_e, wl_ref[e], preferred_element_type=_F32, precision=lax.Precision.HIGHEST) \
        + bl_ref[e][None, :]
  x1 = jnp.maximum(acc, 0.0)
  x1_ref[0] = x1[:, :HALF]
  x1_ref[1] = x1[:, HALF:]


def _k2(agg0, cnt, x, wl_stack, bl_stack, wr_stack):
  return pl.pallas_call(
      _k2_body,
      grid=(NB,),
      in_specs=[
          pl.BlockSpec((2, NET, BLK, HALF), lambda i: (0, 0, i, 0)),
          pl.BlockSpec((NET, SC_NS, 2, 1, BLK), lambda i: (0, 0, 0, 0, i)),
          pl.BlockSpec((BLK, D_IN), lambda i: (i, 0)),
          pl.BlockSpec((NET, D_IN, H), lambda i: (0, 0, 0)),
          pl.BlockSpec((NET, H), lambda i: (0, 0)),
          pl.BlockSpec((NET, D_IN, H), lambda i: (0, 0, 0)),
      ],
      out_specs=pl.BlockSpec((2, BLK, HALF), lambda i: (0, i, 0)),
      out_shape=jax.ShapeDtypeStruct((2, N, HALF), _F32),
  )(agg0, cnt, x, wl_stack, bl_stack, wr_stack)


def _k3_body(agg_ref, cnt_ref, x1_ref, wl_ref, bl_ref, wr_ref,
             wlin_ref, blin_ref, y_ref, ps_ref, psq_ref):
  recip = _recip_cnt(cnt_ref)
  x1 = jnp.concatenate([x1_ref[0], x1_ref[1]], axis=1)
  wr_s = wr_ref[0] + wr_ref[1] + wr_ref[2]
  acc = jnp.dot(x1, wr_s, preferred_element_type=_F32, precision=lax.Precision.HIGHEST)
  for e in range(NET):
    agg_e = jnp.concatenate([agg_ref[e, 0], agg_ref[e, 1]], axis=1)
    mean_e = agg_e * recip[e][:, None]
    acc = acc + jnp.dot(mean_e, wl_ref[e], preferred_element_type=_F32, precision=lax.Precision.HIGHEST) \
        + bl_ref[e][None, :]
  x2 = jnp.maximum(acc, 0.0)
  y = jnp.dot(x2, wlin_ref[...], preferred_element_type=_F32, precision=lax.Precision.HIGHEST) + blin_ref[0]
  y_ref[...] = y
  nb = pl.program_id(0)
  row = nb * BLK + lax.broadcasted_iota(jnp.int32, (BLK, 1), 0)
  ym = jnp.where(row < N, y, 0.0)
  ps_ref[0, 0] = jnp.sum(ym, axis=0)
  psq_ref[0, 0] = jnp.sum(ym * ym, axis=0)


def _k3(agg1, cnt, x1b, wl1_stack, bl1_stack, wr1_stack, w_lin, b_lin2):
  return pl.pallas_call(
      _k3_body,
      grid=(NB,),
      in_specs=[
          pl.BlockSpec((NET, 2, BLK, HALF), lambda i: (0, 0, i, 0)),
          pl.BlockSpec((NET, SC_NS, 2, 1, BLK), lambda i: (0, 0, 0, 0, i)),
          pl.BlockSpec((2, BLK, HALF), lambda i: (0, i, 0)),
          pl.BlockSpec((NET, H, H), lambda i: (0, 0, 0)),
          pl.BlockSpec((NET, H), lambda i: (0, 0)),
          pl.BlockSpec((NET, H, H), lambda i: (0, 0, 0)),
          pl.BlockSpec((H, ENC), lambda i: (0, 0)),
          pl.BlockSpec((1, ENC), lambda i: (0, 0)),
      ],
      out_specs=[
          pl.BlockSpec((BLK, ENC), lambda i: (i, 0)),
          pl.BlockSpec((1, 1, ENC), lambda i: (i, 0, 0)),
          pl.BlockSpec((1, 1, ENC), lambda i: (i, 0, 0)),
      ],
      out_shape=[
          jax.ShapeDtypeStruct((N, ENC), _F32),
          jax.ShapeDtypeStruct((NB, 1, ENC), _F32),
          jax.ShapeDtypeStruct((NB, 1, ENC), _F32),
      ],
  )(agg1, cnt, x1b, wl1_stack, bl1_stack, wr1_stack, w_lin, b_lin2)


def _layer_norm_rows(h, g, b):
  hm = jnp.mean(h, axis=1, keepdims=True)
  hv = jnp.mean(h * h, axis=1, keepdims=True) - hm * hm
  return (h - hm) * lax.rsqrt(hv + 1e-5) * g[None, :] + b[None, :]


def _k4_body(y_ref, ps_ref, psq_ref, bng_ref, bnb_ref,
             pw1_ref, pb1_ref, plg_ref, plb_ref, pw2_ref, pb2_ref,
             kw1x_ref, kw1p_ref, kb1_ref, klg_ref, klb_ref,
             kw2_ref, kb2_ref, pc_ref, ks_ref):
  tot = jnp.sum(ps_ref[...], axis=(0, 1))
  tot2 = jnp.sum(psq_ref[...], axis=(0, 1))
  mu = tot / N
  var = tot2 / N - mu * mu
  inv = lax.rsqrt(var + 1e-5)
  y = y_ref[...]
  xb = (y - mu[None, :]) * (inv * bng_ref[0])[None, :] + bnb_ref[0][None, :]
  h = jnp.maximum(jnp.dot(xb, pw1_ref[...], preferred_element_type=_F32, precision=lax.Precision.HIGHEST)
                  + pb1_ref[0][None, :], 0.0)
  h = _layer_norm_rows(h, plg_ref[0], plb_ref[0])
  pc = jnp.dot(h, pw2_ref[...], preferred_element_type=_F32, precision=lax.Precision.HIGHEST) + pb2_ref[0][None, :]
  pc_ref[...] = pc
  h2 = jnp.maximum(jnp.dot(xb, kw1x_ref[...], preferred_element_type=_F32, precision=lax.Precision.HIGHEST)
                   + jnp.dot(pc, kw1p_ref[...], preferred_element_type=_F32, precision=lax.Precision.HIGHEST)
                   + kb1_ref[0][None, :], 0.0)
  h2 = _layer_norm_rows(h2, klg_ref[0], klb_ref[0])
  ks_ref[...] = jnp.dot(h2, kw2_ref[...], preferred_element_type=_F32, precision=lax.Precision.HIGHEST) \
      + kb2_ref[0][None, :]


def _k4(y, ps, psq, args):
  full2 = lambda a: pl.BlockSpec(a.shape, lambda i: tuple(0 for _ in a.shape))
  return pl.pallas_call(
      _k4_body,
      grid=(NB,),
      in_specs=[
          pl.BlockSpec((BLK, ENC), lambda i: (i, 0)),
          pl.BlockSpec((NB, 1, ENC), lambda i: (0, 0, 0)),
          pl.BlockSpec((NB, 1, ENC), lambda i: (0, 0, 0)),
      ] + [full2(a) for a in args],
      out_specs=[
          pl.BlockSpec((BLK, 128), lambda i: (i, 0)),
          pl.BlockSpec((BLK, 128), lambda i: (i, 0)),
      ],
      out_shape=[
          jax.ShapeDtypeStruct((N, 128), _F32),
          jax.ShapeDtypeStruct((N, 128), _F32),
      ],
  )(y, ps, psq, *args)


def kernel(x_note, edge_index_0, edge_index_1, edge_index_2,
           wl0_0, bl0_0, wr0_0, wl0_1, bl0_1, wr0_1, wl0_2, bl0_2, wr0_2,
           wl1_0, bl1_0, wr1_0, wl1_1, bl1_1, wr1_1, wl1_2, bl1_2, wr1_2,
           w_lin, b_lin, bn_gamma, bn_beta,
           pc_w1, pc_b1, pc_ln_g, pc_ln_b, pc_w2, pc_b2,
           ks_w1, ks_b1, ks_ln_g, ks_ln_b, ks_w2, ks_b2):
  edges = (edge_index_0[0], edge_index_1[0], edge_index_2[0],
           edge_index_0[1], edge_index_1[1], edge_index_2[1])
  wl0 = jnp.stack([wl0_0, wl0_1, wl0_2])
  bl0 = jnp.stack([bl0_0, bl0_1, bl0_2])
  wr0 = jnp.stack([wr0_0, wr0_1, wr0_2])
  wl1 = jnp.stack([wl1_0, wl1_1, wl1_2])
  bl1 = jnp.stack([bl1_0, bl1_1, bl1_2])
  wr1 = jnp.stack([wr1_0, wr1_1, wr1_2])
  zeros = jnp.zeros((NPAD, HALF), _F32)

  agg0, cnt = _make_agg(True)(x_note, *edges, zeros)
  x1b = _k2(agg0, cnt, x_note, wl0, bl0, wr0)
  agg1 = _make_agg(False)(x1b, *edges, zeros)[0]
  y, ps, psq = _k3(agg1, cnt, x1b, wl1, bl1, wr1, w_lin, b_lin[None, :])

  pad = lambda a, r, c: jnp.pad(a, ((0, r - a.shape[0]), (0, c - a.shape[1])))
  pad1 = lambda a, c: jnp.pad(a, ((0, c - a.shape[0],)))[None, :]
  head_args = (
      bn_gamma[None, :], bn_beta[None, :],
      pc_w1, pc_b1[None, :], pc_ln_g[None, :], pc_ln_b[None, :],
      pad(pc_w2, 128, 128), pad1(pc_b2, 128),
      ks_w1[:ENC], pad(ks_w1[ENC:], 128, 128), ks_b1[None, :],
      ks_ln_g[None, :], ks_ln_b[None, :],
      pad(ks_w2, 128, 128), pad1(ks_b2, 128),
  )
  pc_p, ks_p = _k4(y, ps, psq, head_args)
  return pc_p[:, :PC], ks_p[:, :KS]
